# baseline (speedup 1.0000x reference)
import functools

import jax
import jax.numpy as jnp
import numpy as np
from jax.experimental import pallas as pl
from jax.experimental.pallas import tpu as pltpu

N = 6000
NPAD = 6144
CI = 1024
CJ = 2048
HID = 16
CUTS = 8
RADIUS = 9.0
SQRT2 = float(np.sqrt(2.0))
EPS_GN = 1e-5


def _leaky(x):
    return jnp.where(x >= 0, x, 0.2 * x)


def _group_norm_cols(f, gamma, beta, count):
    e16_4 = jnp.repeat(jnp.eye(4, dtype=f.dtype), 4, axis=0)
    e4_16 = e16_4.T
    cs = jnp.sum(f, axis=0, keepdims=True)
    cq = jnp.sum(f * f, axis=0, keepdims=True)
    denom = 4.0 * count
    gm = jnp.dot(cs, e16_4) / denom
    gq = jnp.dot(cq, e16_4) / denom
    gv = gq - gm * gm
    m16 = jnp.dot(gm, e4_16)
    v16 = jnp.dot(gv, e4_16)
    return (f - m16) / jnp.sqrt(v16 + EPS_GN) * gamma + beta


def _pre_body(feats_ref, wo1t_ref, bo1_ref, wo2t_ref, bo2_ref,
              wi1t_ref, bi1_ref, wi2t_ref, bi2_ref, gin_ref, bin_ref,
              scores_ref, f_ref):
    feats = feats_ref[...]
    s = _leaky(jnp.dot(feats, wo1t_ref[...]) + bo1_ref[...])
    scores_ref[...] = jnp.dot(s, wo2t_ref[...]) + bo2_ref[...]
    f = _leaky(jnp.dot(feats, wi1t_ref[...]) + bi1_ref[...])
    f = _leaky(jnp.dot(f, wi2t_ref[...]) + bi2_ref[...])
    f_ref[...] = _group_norm_cols(f, gin_ref[...], bin_ref[...], float(N))


def _pass1_body(nj_steps, xyzt_ref, xyzi_ref, xyzj_ref, nrmt_ref, nrmi_ref,
                wt_ref, m924_ref, c_ref, acc_ref):
    j = pl.program_id(1)

    @pl.when(j == 0)
    def _():
        acc_ref[...] = jnp.zeros_like(acc_ref)

    inv = 1.0 / RADIUS
    pxj = xyzt_ref[0:1, :] * inv
    pyj = xyzt_ref[1:2, :] * inv
    pzj = xyzt_ref[2:3, :] * inv
    pxi = xyzi_ref[:, 0:1] * inv
    pyi = xyzi_ref[:, 1:2] * inv
    pzi = xyzi_ref[:, 2:3] * inv
    dx = pxj - pxi
    dy = pyj - pyi
    dz = pzj - pzi
    d2 = dx * dx + dy * dy + dz * dz
    cos = jnp.dot(nrmi_ref[...], nrmt_ref[...],
                  preferred_element_type=jnp.float32)
    t = 2.0 - cos
    win = jnp.exp(-d2 * t * t) * wt_ref[...]
    pj = xyzj_ref[...] * inv
    pj4 = jnp.concatenate([pj, jnp.ones((pj.shape[0], 1), jnp.float32)],
                          axis=1)
    acc_ref[...] += jnp.dot(win, pj4, preferred_element_type=jnp.float32)

    @pl.when(j == nj_steps - 1)
    def _():
        nrm = nrmi_ref[...]
        x = nrm[:, 0:1]
        y = nrm[:, 1:2]
        z = nrm[:, 2:3]
        sgn = 2.0 * (z >= 0).astype(jnp.float32) - 1.0
        a = -1.0 / (sgn + z)
        b = x * y * a
        u = jnp.concatenate([1.0 + sgn * x * x * a, sgn * b, -sgn * x], axis=1)
        v = jnp.concatenate([b, sgn + y * y * a, -y], axis=1)
        pi = xyzi_ref[...] * inv
        vv = acc_ref[:, 0:3] - pi * acc_ref[:, 3:4]
        ov0 = jnp.sum(u * vv, axis=1, keepdims=True) + 1e-5
        ov1 = jnp.sum(v * vv, axis=1, keepdims=True) + 1e-5
        inv_n = 1.0 / jnp.maximum(jnp.sqrt(ov0 * ov0 + ov1 * ov1), 1e-12)
        ex = ov0 * inv_n
        ey = ov1 * inv_n
        u2 = ex * u + ey * v
        v2 = -ey * u + ex * v
        bas9 = jnp.concatenate([nrm, u2, v2], axis=1)
        c_ref[...] = jnp.dot(bas9, m924_ref[...],
                             preferred_element_type=jnp.float32)


def _pass2_body(nj_steps, xyzt_ref, xyzi_ref, nrmt_ref, nrmi_ref, c_ref,
                f_ref, b1_ref, a2t_ref, b2_ref, wn1t_ref, bn1_ref,
                wn2t_ref, bn2_ref, y_ref, acc_ref):
    j = pl.program_id(1)

    @pl.when(j == 0)
    def _():
        acc_ref[...] = jnp.zeros_like(acc_ref)

    inv = 1.0 / (SQRT2 * RADIUS)
    pxj = xyzt_ref[0:1, :] * inv
    pyj = xyzt_ref[1:2, :] * inv
    pzj = xyzt_ref[2:3, :] * inv
    pxi = xyzi_ref[:, 0:1] * inv
    pyi = xyzi_ref[:, 1:2] * inv
    pzi = xyzi_ref[:, 2:3] * inv
    dx = pxj - pxi
    dy = pyj - pyi
    dz = pzj - pzi
    d2 = dx * dx + dy * dy + dz * dz
    cos = jnp.dot(nrmi_ref[...], nrmt_ref[...],
                  preferred_element_type=jnp.float32)
    t = 2.0 - cos
    win = jnp.exp(-d2 * t * t)
    fb = f_ref[...]
    acc = acc_ref[...]
    acc = acc + (jnp.dot(win, fb,
                         preferred_element_type=jnp.float32) * b2_ref[...])
    for c in range(CUTS):
        cx = c_ref[:, 3 * c:3 * c + 1]
        cy = c_ref[:, 3 * c + 1:3 * c + 2]
        cz = c_ref[:, 3 * c + 2:3 * c + 3]
        p = dx * cx + dy * cy + dz * cz + b1_ref[0:1, c:c + 1]
        wc = win * jnp.maximum(p, 0.0)
        acc = acc + (jnp.dot(wc, fb, preferred_element_type=jnp.float32)
                     * a2t_ref[c:c + 1, :])
    acc_ref[...] = acc

    @pl.when(j == nj_steps - 1)
    def _():
        out = _leaky(jnp.dot(acc_ref[...], wn1t_ref[...]) + bn1_ref[...])
        out = _leaky(jnp.dot(out, wn2t_ref[...]) + bn2_ref[...])
        y_ref[...] = out


def _post_body(y_ref, feats_ref, gout_ref, bout_ref, wl1t_ref, bl1_ref,
               wl2t_ref, bl2_ref, wtt_ref, bt_ref, out_ref):
    yn = _group_norm_cols(y_ref[...], gout_ref[...], bout_ref[...], float(N))
    h = jnp.maximum(jnp.dot(yn, wl1t_ref[...]) + bl1_ref[...], 0.0)
    h = jnp.dot(h, wl2t_ref[...]) + bl2_ref[...]
    out_ref[...] = jnp.dot(feats_ref[...], wtt_ref[...]) + bt_ref[...] + h


def _pad_rows(x, value):
    return jnp.pad(x, ((0, NPAD - x.shape[0]), (0, 0)), constant_values=value)


@jax.jit
def kernel(surface_xyz, surface_normals, features, Wo1, bo1, Wo2, bo2,
           Wi1, bi1, Wi2, bi2, g_in, b_in, A1, B1, A2, B2,
           Wn1, bn1, Wn2, bn2, g_out, b_out, Wl1, bl1, Wl2, bl2, Wt, bt):
    f32 = jnp.float32
    row = lambda b: b.reshape(1, -1)

    scores, f = pl.pallas_call(
        _pre_body,
        out_shape=(jax.ShapeDtypeStruct((N, 1), f32),
                   jax.ShapeDtypeStruct((N, HID), f32)),
    )(features, Wo1.T, row(bo1), Wo2.T, row(bo2),
      Wi1.T, row(bi1), Wi2.T, row(bi2), row(g_in), row(b_in))

    xyz_p = _pad_rows(surface_xyz, 1e6)
    nrm_p = _pad_rows(surface_normals, 0.0)
    f_p = _pad_rows(f, 0.0)
    w_p = _pad_rows(scores, 0.0)
    xyz_t = xyz_p.T
    nrm_t = nrm_p.T
    w_t = w_p.T

    ni = NPAD // CI
    nj = NPAD // CJ

    m924 = jnp.zeros((9, 3 * CUTS), f32)
    for k in range(3):
        for d in range(3):
            m924 = m924.at[3 * k + d, d::3].set(A1[:, k])

    c24 = pl.pallas_call(
        functools.partial(_pass1_body, nj),
        grid=(ni, nj),
        in_specs=[
            pl.BlockSpec((3, CJ), lambda i, j: (0, j)),
            pl.BlockSpec((CI, 3), lambda i, j: (i, 0)),
            pl.BlockSpec((CJ, 3), lambda i, j: (j, 0)),
            pl.BlockSpec((3, CJ), lambda i, j: (0, j)),
            pl.BlockSpec((CI, 3), lambda i, j: (i, 0)),
            pl.BlockSpec((1, CJ), lambda i, j: (0, j)),
            pl.BlockSpec((9, 3 * CUTS), lambda i, j: (0, 0)),
        ],
        out_specs=pl.BlockSpec((CI, 3 * CUTS), lambda i, j: (i, 0)),
        out_shape=jax.ShapeDtypeStruct((NPAD, 3 * CUTS), f32),
        scratch_shapes=[pltpu.VMEM((CI, 4), f32)],
        compiler_params=pltpu.CompilerParams(
            dimension_semantics=("arbitrary", "arbitrary")),
    )(xyz_t, xyz_p, xyz_p, nrm_t, nrm_p, w_t, m924)

    y = pl.pallas_call(
        functools.partial(_pass2_body, nj),
        grid=(ni, nj),
        in_specs=[
            pl.BlockSpec((3, CJ), lambda i, j: (0, j)),
            pl.BlockSpec((CI, 3), lambda i, j: (i, 0)),
            pl.BlockSpec((3, CJ), lambda i, j: (0, j)),
            pl.BlockSpec((CI, 3), lambda i, j: (i, 0)),
            pl.BlockSpec((CI, 3 * CUTS), lambda i, j: (i, 0)),
            pl.BlockSpec((CJ, HID), lambda i, j: (j, 0)),
            pl.BlockSpec((1, CUTS), lambda i, j: (0, 0)),
            pl.BlockSpec((CUTS, HID), lambda i, j: (0, 0)),
            pl.BlockSpec((1, HID), lambda i, j: (0, 0)),
            pl.BlockSpec((HID, HID), lambda i, j: (0, 0)),
            pl.BlockSpec((1, HID), lambda i, j: (0, 0)),
            pl.BlockSpec((HID, HID), lambda i, j: (0, 0)),
            pl.BlockSpec((1, HID), lambda i, j: (0, 0)),
        ],
        out_specs=pl.BlockSpec((CI, HID), lambda i, j: (i, 0)),
        out_shape=jax.ShapeDtypeStruct((NPAD, HID), f32),
        scratch_shapes=[pltpu.VMEM((CI, HID), f32)],
        compiler_params=pltpu.CompilerParams(
            dimension_semantics=("arbitrary", "arbitrary")),
    )(xyz_t, xyz_p, nrm_t, nrm_p, c24, f_p, row(B1), A2[:HID].T,
      row(B2[:HID]), Wn1.T, row(bn1), Wn2.T, row(bn2))

    out = pl.pallas_call(
        _post_body,
        out_shape=jax.ShapeDtypeStruct((N, HID), f32),
    )(y[:N], features, row(g_out), row(b_out),
      Wl1.T, row(bl1), Wl2.T, row(bl2), Wt.T, row(bt))
    return out


# speedup vs baseline: 1.0220x; 1.0220x over previous
import functools

import jax
import jax.numpy as jnp
import numpy as np
from jax.experimental import pallas as pl
from jax.experimental.pallas import tpu as pltpu

N = 6000
NPAD = 6144
CI = 512
CJ = 3072
HID = 16
CUTS = 8
RADIUS = 9.0
SQRT2 = float(np.sqrt(2.0))
EPS_GN = 1e-5


def _leaky(x):
    return jnp.where(x >= 0, x, 0.2 * x)


def _group_norm_cols(f, gamma, beta, count):
    e16_4 = jnp.repeat(jnp.eye(4, dtype=f.dtype), 4, axis=0)
    e4_16 = e16_4.T
    cs = jnp.sum(f, axis=0, keepdims=True)
    cq = jnp.sum(f * f, axis=0, keepdims=True)
    denom = 4.0 * count
    gm = jnp.dot(cs, e16_4) / denom
    gq = jnp.dot(cq, e16_4) / denom
    gv = gq - gm * gm
    m16 = jnp.dot(gm, e4_16)
    v16 = jnp.dot(gv, e4_16)
    return (f - m16) / jnp.sqrt(v16 + EPS_GN) * gamma + beta


def _pre_body(feats_ref, wo1t_ref, bo1_ref, wo2t_ref, bo2_ref,
              wi1t_ref, bi1_ref, wi2t_ref, bi2_ref, gin_ref, bin_ref,
              scores_ref, f_ref):
    feats = feats_ref[...]
    s = _leaky(jnp.dot(feats, wo1t_ref[...]) + bo1_ref[...])
    scores_ref[...] = jnp.dot(s, wo2t_ref[...]) + bo2_ref[...]
    f = _leaky(jnp.dot(feats, wi1t_ref[...]) + bi1_ref[...])
    f = _leaky(jnp.dot(f, wi2t_ref[...]) + bi2_ref[...])
    f_ref[...] = _group_norm_cols(f, gin_ref[...], bin_ref[...], float(N))


def _pass1_body(nj_steps, xyzt_ref, xyzi_ref, xyzj_ref, nrmt_ref, nrmi_ref,
                wt_ref, m924_ref, c_ref, acc_ref):
    j = pl.program_id(1)

    @pl.when(j == 0)
    def _():
        acc_ref[...] = jnp.zeros_like(acc_ref)

    inv = 1.0 / RADIUS
    pxj = xyzt_ref[0:1, :] * inv
    pyj = xyzt_ref[1:2, :] * inv
    pzj = xyzt_ref[2:3, :] * inv
    pxi = xyzi_ref[:, 0:1] * inv
    pyi = xyzi_ref[:, 1:2] * inv
    pzi = xyzi_ref[:, 2:3] * inv
    dx = pxj - pxi
    dy = pyj - pyi
    dz = pzj - pzi
    d2 = dx * dx + dy * dy + dz * dz
    cos = jnp.dot(nrmi_ref[...], nrmt_ref[...],
                  preferred_element_type=jnp.float32)
    t = 2.0 - cos
    win = jnp.exp(-d2 * t * t) * wt_ref[...]
    pj = xyzj_ref[...] * inv
    pj4 = jnp.concatenate([pj, jnp.ones((pj.shape[0], 1), jnp.float32)],
                          axis=1)
    acc_ref[...] += jnp.dot(win, pj4, preferred_element_type=jnp.float32)

    @pl.when(j == nj_steps - 1)
    def _():
        nrm = nrmi_ref[...]
        x = nrm[:, 0:1]
        y = nrm[:, 1:2]
        z = nrm[:, 2:3]
        sgn = 2.0 * (z >= 0).astype(jnp.float32) - 1.0
        a = -1.0 / (sgn + z)
        b = x * y * a
        u = jnp.concatenate([1.0 + sgn * x * x * a, sgn * b, -sgn * x], axis=1)
        v = jnp.concatenate([b, sgn + y * y * a, -y], axis=1)
        pi = xyzi_ref[...] * inv
        vv = acc_ref[:, 0:3] - pi * acc_ref[:, 3:4]
        ov0 = jnp.sum(u * vv, axis=1, keepdims=True) + 1e-5
        ov1 = jnp.sum(v * vv, axis=1, keepdims=True) + 1e-5
        inv_n = 1.0 / jnp.maximum(jnp.sqrt(ov0 * ov0 + ov1 * ov1), 1e-12)
        ex = ov0 * inv_n
        ey = ov1 * inv_n
        u2 = ex * u + ey * v
        v2 = -ey * u + ex * v
        bas9 = jnp.concatenate([nrm, u2, v2], axis=1)
        c_ref[...] = jnp.dot(bas9, m924_ref[...],
                             preferred_element_type=jnp.float32)


def _pass2_body(nj_steps, xyzt_ref, xyzi_ref, nrmt_ref, nrmi_ref, c_ref,
                f_ref, b1_ref, a2t_ref, b2_ref, wn1t_ref, bn1_ref,
                wn2t_ref, bn2_ref, y_ref, acc_ref):
    j = pl.program_id(1)

    @pl.when(j == 0)
    def _():
        acc_ref[...] = jnp.zeros_like(acc_ref)

    inv = 1.0 / (SQRT2 * RADIUS)
    pxj = xyzt_ref[0:1, :] * inv
    pyj = xyzt_ref[1:2, :] * inv
    pzj = xyzt_ref[2:3, :] * inv
    pxi = xyzi_ref[:, 0:1] * inv
    pyi = xyzi_ref[:, 1:2] * inv
    pzi = xyzi_ref[:, 2:3] * inv
    dx = pxj - pxi
    dy = pyj - pyi
    dz = pzj - pzi
    d2 = dx * dx + dy * dy + dz * dz
    cos = jnp.dot(nrmi_ref[...], nrmt_ref[...],
                  preferred_element_type=jnp.float32)
    t = 2.0 - cos
    win = jnp.exp(-d2 * t * t)
    fb = f_ref[...]
    acc = acc_ref[...]
    acc = acc + (jnp.dot(win, fb,
                         preferred_element_type=jnp.float32) * b2_ref[...])
    for c in range(CUTS):
        cx = c_ref[:, 3 * c:3 * c + 1]
        cy = c_ref[:, 3 * c + 1:3 * c + 2]
        cz = c_ref[:, 3 * c + 2:3 * c + 3]
        p = dx * cx + dy * cy + dz * cz + b1_ref[0:1, c:c + 1]
        wc = win * jnp.maximum(p, 0.0)
        acc = acc + (jnp.dot(wc, fb, preferred_element_type=jnp.float32)
                     * a2t_ref[c:c + 1, :])
    acc_ref[...] = acc

    @pl.when(j == nj_steps - 1)
    def _():
        out = _leaky(jnp.dot(acc_ref[...], wn1t_ref[...]) + bn1_ref[...])
        out = _leaky(jnp.dot(out, wn2t_ref[...]) + bn2_ref[...])
        y_ref[...] = out


def _post_body(y_ref, feats_ref, gout_ref, bout_ref, wl1t_ref, bl1_ref,
               wl2t_ref, bl2_ref, wtt_ref, bt_ref, out_ref):
    yn = _group_norm_cols(y_ref[...], gout_ref[...], bout_ref[...], float(N))
    h = jnp.maximum(jnp.dot(yn, wl1t_ref[...]) + bl1_ref[...], 0.0)
    h = jnp.dot(h, wl2t_ref[...]) + bl2_ref[...]
    out_ref[...] = jnp.dot(feats_ref[...], wtt_ref[...]) + bt_ref[...] + h


def _pad_rows(x, value):
    return jnp.pad(x, ((0, NPAD - x.shape[0]), (0, 0)), constant_values=value)


@jax.jit
def kernel(surface_xyz, surface_normals, features, Wo1, bo1, Wo2, bo2,
           Wi1, bi1, Wi2, bi2, g_in, b_in, A1, B1, A2, B2,
           Wn1, bn1, Wn2, bn2, g_out, b_out, Wl1, bl1, Wl2, bl2, Wt, bt):
    f32 = jnp.float32
    row = lambda b: b.reshape(1, -1)

    scores, f = pl.pallas_call(
        _pre_body,
        out_shape=(jax.ShapeDtypeStruct((N, 1), f32),
                   jax.ShapeDtypeStruct((N, HID), f32)),
    )(features, Wo1.T, row(bo1), Wo2.T, row(bo2),
      Wi1.T, row(bi1), Wi2.T, row(bi2), row(g_in), row(b_in))

    xyz_p = _pad_rows(surface_xyz, 1e6)
    nrm_p = _pad_rows(surface_normals, 0.0)
    f_p = _pad_rows(f, 0.0)
    w_p = _pad_rows(scores, 0.0)
    xyz_t = xyz_p.T
    nrm_t = nrm_p.T
    w_t = w_p.T

    ni = NPAD // CI
    nj = NPAD // CJ

    m924 = jnp.zeros((9, 3 * CUTS), f32)
    for k in range(3):
        for d in range(3):
            m924 = m924.at[3 * k + d, d::3].set(A1[:, k])

    c24 = pl.pallas_call(
        functools.partial(_pass1_body, nj),
        grid=(ni, nj),
        in_specs=[
            pl.BlockSpec((3, CJ), lambda i, j: (0, j)),
            pl.BlockSpec((CI, 3), lambda i, j: (i, 0)),
            pl.BlockSpec((CJ, 3), lambda i, j: (j, 0)),
            pl.BlockSpec((3, CJ), lambda i, j: (0, j)),
            pl.BlockSpec((CI, 3), lambda i, j: (i, 0)),
            pl.BlockSpec((1, CJ), lambda i, j: (0, j)),
            pl.BlockSpec((9, 3 * CUTS), lambda i, j: (0, 0)),
        ],
        out_specs=pl.BlockSpec((CI, 3 * CUTS), lambda i, j: (i, 0)),
        out_shape=jax.ShapeDtypeStruct((NPAD, 3 * CUTS), f32),
        scratch_shapes=[pltpu.VMEM((CI, 4), f32)],
        compiler_params=pltpu.CompilerParams(
            dimension_semantics=("arbitrary", "arbitrary")),
    )(xyz_t, xyz_p, xyz_p, nrm_t, nrm_p, w_t, m924)

    y = pl.pallas_call(
        functools.partial(_pass2_body, nj),
        grid=(ni, nj),
        in_specs=[
            pl.BlockSpec((3, CJ), lambda i, j: (0, j)),
            pl.BlockSpec((CI, 3), lambda i, j: (i, 0)),
            pl.BlockSpec((3, CJ), lambda i, j: (0, j)),
            pl.BlockSpec((CI, 3), lambda i, j: (i, 0)),
            pl.BlockSpec((CI, 3 * CUTS), lambda i, j: (i, 0)),
            pl.BlockSpec((CJ, HID), lambda i, j: (j, 0)),
            pl.BlockSpec((1, CUTS), lambda i, j: (0, 0)),
            pl.BlockSpec((CUTS, HID), lambda i, j: (0, 0)),
            pl.BlockSpec((1, HID), lambda i, j: (0, 0)),
            pl.BlockSpec((HID, HID), lambda i, j: (0, 0)),
            pl.BlockSpec((1, HID), lambda i, j: (0, 0)),
            pl.BlockSpec((HID, HID), lambda i, j: (0, 0)),
            pl.BlockSpec((1, HID), lambda i, j: (0, 0)),
        ],
        out_specs=pl.BlockSpec((CI, HID), lambda i, j: (i, 0)),
        out_shape=jax.ShapeDtypeStruct((NPAD, HID), f32),
        scratch_shapes=[pltpu.VMEM((CI, HID), f32)],
        compiler_params=pltpu.CompilerParams(
            dimension_semantics=("arbitrary", "arbitrary")),
    )(xyz_t, xyz_p, nrm_t, nrm_p, c24, f_p, row(B1), A2[:HID].T,
      row(B2[:HID]), Wn1.T, row(bn1), Wn2.T, row(bn2))

    out = pl.pallas_call(
        _post_body,
        out_shape=jax.ShapeDtypeStruct((N, HID), f32),
    )(y[:N], features, row(g_out), row(b_out),
      Wl1.T, row(bl1), Wl2.T, row(bl2), Wt.T, row(bt))
    return out


# speedup vs baseline: 1.0243x; 1.0022x over previous
import functools

import jax
import jax.numpy as jnp
import numpy as np
from jax.experimental import pallas as pl
from jax.experimental.pallas import tpu as pltpu

N = 6000
NPAD = 6144
CI = 256
CJ = 6144
HID = 16
CUTS = 8
RADIUS = 9.0
SQRT2 = float(np.sqrt(2.0))
EPS_GN = 1e-5


def _leaky(x):
    return jnp.where(x >= 0, x, 0.2 * x)


def _group_norm_cols(f, gamma, beta, count):
    e16_4 = jnp.repeat(jnp.eye(4, dtype=f.dtype), 4, axis=0)
    e4_16 = e16_4.T
    cs = jnp.sum(f, axis=0, keepdims=True)
    cq = jnp.sum(f * f, axis=0, keepdims=True)
    denom = 4.0 * count
    gm = jnp.dot(cs, e16_4) / denom
    gq = jnp.dot(cq, e16_4) / denom
    gv = gq - gm * gm
    m16 = jnp.dot(gm, e4_16)
    v16 = jnp.dot(gv, e4_16)
    return (f - m16) / jnp.sqrt(v16 + EPS_GN) * gamma + beta


def _pre_body(feats_ref, wo1t_ref, bo1_ref, wo2t_ref, bo2_ref,
              wi1t_ref, bi1_ref, wi2t_ref, bi2_ref, gin_ref, bin_ref,
              scores_ref, f_ref):
    feats = feats_ref[...]
    s = _leaky(jnp.dot(feats, wo1t_ref[...]) + bo1_ref[...])
    scores_ref[...] = jnp.dot(s, wo2t_ref[...]) + bo2_ref[...]
    f = _leaky(jnp.dot(feats, wi1t_ref[...]) + bi1_ref[...])
    f = _leaky(jnp.dot(f, wi2t_ref[...]) + bi2_ref[...])
    f_ref[...] = _group_norm_cols(f, gin_ref[...], bin_ref[...], float(N))


def _pass1_body(nj_steps, xyzt_ref, xyzi_ref, xyzj_ref, nrmt_ref, nrmi_ref,
                wt_ref, m924_ref, c_ref, acc_ref):
    j = pl.program_id(1)

    @pl.when(j == 0)
    def _():
        acc_ref[...] = jnp.zeros_like(acc_ref)

    inv = 1.0 / RADIUS
    pxj = xyzt_ref[0:1, :] * inv
    pyj = xyzt_ref[1:2, :] * inv
    pzj = xyzt_ref[2:3, :] * inv
    pxi = xyzi_ref[:, 0:1] * inv
    pyi = xyzi_ref[:, 1:2] * inv
    pzi = xyzi_ref[:, 2:3] * inv
    dx = pxj - pxi
    dy = pyj - pyi
    dz = pzj - pzi
    d2 = dx * dx + dy * dy + dz * dz
    cos = jnp.dot(nrmi_ref[...], nrmt_ref[...],
                  preferred_element_type=jnp.float32)
    t = 2.0 - cos
    win = jnp.exp(-d2 * t * t) * wt_ref[...]
    pj = xyzj_ref[...] * inv
    pj4 = jnp.concatenate([pj, jnp.ones((pj.shape[0], 1), jnp.float32)],
                          axis=1)
    acc_ref[...] += jnp.dot(win, pj4, preferred_element_type=jnp.float32)

    @pl.when(j == nj_steps - 1)
    def _():
        nrm = nrmi_ref[...]
        x = nrm[:, 0:1]
        y = nrm[:, 1:2]
        z = nrm[:, 2:3]
        sgn = 2.0 * (z >= 0).astype(jnp.float32) - 1.0
        a = -1.0 / (sgn + z)
        b = x * y * a
        u = jnp.concatenate([1.0 + sgn * x * x * a, sgn * b, -sgn * x], axis=1)
        v = jnp.concatenate([b, sgn + y * y * a, -y], axis=1)
        pi = xyzi_ref[...] * inv
        vv = acc_ref[:, 0:3] - pi * acc_ref[:, 3:4]
        ov0 = jnp.sum(u * vv, axis=1, keepdims=True) + 1e-5
        ov1 = jnp.sum(v * vv, axis=1, keepdims=True) + 1e-5
        inv_n = 1.0 / jnp.maximum(jnp.sqrt(ov0 * ov0 + ov1 * ov1), 1e-12)
        ex = ov0 * inv_n
        ey = ov1 * inv_n
        u2 = ex * u + ey * v
        v2 = -ey * u + ex * v
        bas9 = jnp.concatenate([nrm, u2, v2], axis=1)
        c_ref[...] = jnp.dot(bas9, m924_ref[...],
                             preferred_element_type=jnp.float32)


def _pass2_body(nj_steps, xyzt_ref, xyzi_ref, nrmt_ref, nrmi_ref, c_ref,
                f_ref, b1_ref, a2t_ref, b2_ref, wn1t_ref, bn1_ref,
                wn2t_ref, bn2_ref, y_ref, acc_ref):
    j = pl.program_id(1)

    @pl.when(j == 0)
    def _():
        acc_ref[...] = jnp.zeros_like(acc_ref)

    inv = 1.0 / (SQRT2 * RADIUS)
    pxj = xyzt_ref[0:1, :] * inv
    pyj = xyzt_ref[1:2, :] * inv
    pzj = xyzt_ref[2:3, :] * inv
    pxi = xyzi_ref[:, 0:1] * inv
    pyi = xyzi_ref[:, 1:2] * inv
    pzi = xyzi_ref[:, 2:3] * inv
    dx = pxj - pxi
    dy = pyj - pyi
    dz = pzj - pzi
    d2 = dx * dx + dy * dy + dz * dz
    cos = jnp.dot(nrmi_ref[...], nrmt_ref[...],
                  preferred_element_type=jnp.float32)
    t = 2.0 - cos
    win = jnp.exp(-d2 * t * t)
    fb = f_ref[...]
    acc = acc_ref[...]
    acc = acc + (jnp.dot(win, fb,
                         preferred_element_type=jnp.float32) * b2_ref[...])
    for c in range(CUTS):
        cx = c_ref[:, 3 * c:3 * c + 1]
        cy = c_ref[:, 3 * c + 1:3 * c + 2]
        cz = c_ref[:, 3 * c + 2:3 * c + 3]
        p = dx * cx + dy * cy + dz * cz + b1_ref[0:1, c:c + 1]
        wc = win * jnp.maximum(p, 0.0)
        acc = acc + (jnp.dot(wc, fb, preferred_element_type=jnp.float32)
                     * a2t_ref[c:c + 1, :])
    acc_ref[...] = acc

    @pl.when(j == nj_steps - 1)
    def _():
        out = _leaky(jnp.dot(acc_ref[...], wn1t_ref[...]) + bn1_ref[...])
        out = _leaky(jnp.dot(out, wn2t_ref[...]) + bn2_ref[...])
        y_ref[...] = out


def _post_body(y_ref, feats_ref, gout_ref, bout_ref, wl1t_ref, bl1_ref,
               wl2t_ref, bl2_ref, wtt_ref, bt_ref, out_ref):
    yn = _group_norm_cols(y_ref[...], gout_ref[...], bout_ref[...], float(N))
    h = jnp.maximum(jnp.dot(yn, wl1t_ref[...]) + bl1_ref[...], 0.0)
    h = jnp.dot(h, wl2t_ref[...]) + bl2_ref[...]
    out_ref[...] = jnp.dot(feats_ref[...], wtt_ref[...]) + bt_ref[...] + h


def _pad_rows(x, value):
    return jnp.pad(x, ((0, NPAD - x.shape[0]), (0, 0)), constant_values=value)


@jax.jit
def kernel(surface_xyz, surface_normals, features, Wo1, bo1, Wo2, bo2,
           Wi1, bi1, Wi2, bi2, g_in, b_in, A1, B1, A2, B2,
           Wn1, bn1, Wn2, bn2, g_out, b_out, Wl1, bl1, Wl2, bl2, Wt, bt):
    f32 = jnp.float32
    row = lambda b: b.reshape(1, -1)

    scores, f = pl.pallas_call(
        _pre_body,
        out_shape=(jax.ShapeDtypeStruct((N, 1), f32),
                   jax.ShapeDtypeStruct((N, HID), f32)),
    )(features, Wo1.T, row(bo1), Wo2.T, row(bo2),
      Wi1.T, row(bi1), Wi2.T, row(bi2), row(g_in), row(b_in))

    xyz_p = _pad_rows(surface_xyz, 1e6)
    nrm_p = _pad_rows(surface_normals, 0.0)
    f_p = _pad_rows(f, 0.0)
    w_p = _pad_rows(scores, 0.0)
    xyz_t = xyz_p.T
    nrm_t = nrm_p.T
    w_t = w_p.T

    ni = NPAD // CI
    nj = NPAD // CJ

    m924 = jnp.zeros((9, 3 * CUTS), f32)
    for k in range(3):
        for d in range(3):
            m924 = m924.at[3 * k + d, d::3].set(A1[:, k])

    c24 = pl.pallas_call(
        functools.partial(_pass1_body, nj),
        grid=(ni, nj),
        in_specs=[
            pl.BlockSpec((3, CJ), lambda i, j: (0, j)),
            pl.BlockSpec((CI, 3), lambda i, j: (i, 0)),
            pl.BlockSpec((CJ, 3), lambda i, j: (j, 0)),
            pl.BlockSpec((3, CJ), lambda i, j: (0, j)),
            pl.BlockSpec((CI, 3), lambda i, j: (i, 0)),
            pl.BlockSpec((1, CJ), lambda i, j: (0, j)),
            pl.BlockSpec((9, 3 * CUTS), lambda i, j: (0, 0)),
        ],
        out_specs=pl.BlockSpec((CI, 3 * CUTS), lambda i, j: (i, 0)),
        out_shape=jax.ShapeDtypeStruct((NPAD, 3 * CUTS), f32),
        scratch_shapes=[pltpu.VMEM((CI, 4), f32)],
        compiler_params=pltpu.CompilerParams(
            dimension_semantics=("arbitrary", "arbitrary")),
    )(xyz_t, xyz_p, xyz_p, nrm_t, nrm_p, w_t, m924)

    y = pl.pallas_call(
        functools.partial(_pass2_body, nj),
        grid=(ni, nj),
        in_specs=[
            pl.BlockSpec((3, CJ), lambda i, j: (0, j)),
            pl.BlockSpec((CI, 3), lambda i, j: (i, 0)),
            pl.BlockSpec((3, CJ), lambda i, j: (0, j)),
            pl.BlockSpec((CI, 3), lambda i, j: (i, 0)),
            pl.BlockSpec((CI, 3 * CUTS), lambda i, j: (i, 0)),
            pl.BlockSpec((CJ, HID), lambda i, j: (j, 0)),
            pl.BlockSpec((1, CUTS), lambda i, j: (0, 0)),
            pl.BlockSpec((CUTS, HID), lambda i, j: (0, 0)),
            pl.BlockSpec((1, HID), lambda i, j: (0, 0)),
            pl.BlockSpec((HID, HID), lambda i, j: (0, 0)),
            pl.BlockSpec((1, HID), lambda i, j: (0, 0)),
            pl.BlockSpec((HID, HID), lambda i, j: (0, 0)),
            pl.BlockSpec((1, HID), lambda i, j: (0, 0)),
        ],
        out_specs=pl.BlockSpec((CI, HID), lambda i, j: (i, 0)),
        out_shape=jax.ShapeDtypeStruct((NPAD, HID), f32),
        scratch_shapes=[pltpu.VMEM((CI, HID), f32)],
        compiler_params=pltpu.CompilerParams(
            dimension_semantics=("arbitrary", "arbitrary")),
    )(xyz_t, xyz_p, nrm_t, nrm_p, c24, f_p, row(B1), A2[:HID].T,
      row(B2[:HID]), Wn1.T, row(bn1), Wn2.T, row(bn2))

    out = pl.pallas_call(
        _post_body,
        out_shape=jax.ShapeDtypeStruct((N, HID), f32),
    )(y[:N], features, row(g_out), row(b_out),
      Wl1.T, row(bl1), Wl2.T, row(bl2), Wt.T, row(bt))
    return out


# speedup vs baseline: 1.2370x; 1.2077x over previous
import functools

import jax
import jax.numpy as jnp
import numpy as np
from jax.experimental import pallas as pl
from jax.experimental.pallas import tpu as pltpu

N = 6000
NPAD = 6144
CI = 256
CJ = 6144
HID = 16
CUTS = 8
RADIUS = 9.0
SQRT2 = float(np.sqrt(2.0))
EPS_GN = 1e-5


def _leaky(x):
    return jnp.where(x >= 0, x, 0.2 * x)


def _group_norm_cols(f, gamma, beta, count):
    e16_4 = jnp.repeat(jnp.eye(4, dtype=f.dtype), 4, axis=0)
    e4_16 = e16_4.T
    cs = jnp.sum(f, axis=0, keepdims=True)
    cq = jnp.sum(f * f, axis=0, keepdims=True)
    denom = 4.0 * count
    gm = jnp.dot(cs, e16_4) / denom
    gq = jnp.dot(cq, e16_4) / denom
    gv = gq - gm * gm
    m16 = jnp.dot(gm, e4_16)
    v16 = jnp.dot(gv, e4_16)
    return (f - m16) / jnp.sqrt(v16 + EPS_GN) * gamma + beta


def _pre_body(feats_ref, wo1t_ref, bo1_ref, wo2t_ref, bo2_ref,
              wi1t_ref, bi1_ref, wi2t_ref, bi2_ref, gin_ref, bin_ref,
              scores_ref, f_ref):
    feats = feats_ref[...]
    s = _leaky(jnp.dot(feats, wo1t_ref[...]) + bo1_ref[...])
    scores_ref[...] = jnp.dot(s, wo2t_ref[...]) + bo2_ref[...]
    f = _leaky(jnp.dot(feats, wi1t_ref[...]) + bi1_ref[...])
    f = _leaky(jnp.dot(f, wi2t_ref[...]) + bi2_ref[...])
    f_ref[...] = _group_norm_cols(f, gin_ref[...], bin_ref[...], float(N))


def _pass1_body(nj_steps, xyzt_ref, xyzi_ref, xyzj_ref, nrmt_ref, nrmi_ref,
                wt_ref, m924_ref, c_ref, acc_ref):
    j = pl.program_id(1)

    @pl.when(j == 0)
    def _():
        acc_ref[...] = jnp.zeros_like(acc_ref)

    inv = 1.0 / RADIUS
    pxj = xyzt_ref[0:1, :] * inv
    pyj = xyzt_ref[1:2, :] * inv
    pzj = xyzt_ref[2:3, :] * inv
    pxi = xyzi_ref[:, 0:1] * inv
    pyi = xyzi_ref[:, 1:2] * inv
    pzi = xyzi_ref[:, 2:3] * inv
    dx = pxj - pxi
    dy = pyj - pyi
    dz = pzj - pzi
    d2 = dx * dx + dy * dy + dz * dz
    cos = jnp.dot(nrmi_ref[...], nrmt_ref[...],
                  preferred_element_type=jnp.float32)
    t = 2.0 - cos
    win = jnp.exp(-d2 * t * t) * wt_ref[...]
    pj = xyzj_ref[...] * inv
    pj4 = jnp.concatenate([pj, jnp.ones((pj.shape[0], 1), jnp.float32)],
                          axis=1)
    acc_ref[...] += jnp.dot(win, pj4, preferred_element_type=jnp.float32)

    @pl.when(j == nj_steps - 1)
    def _():
        nrm = nrmi_ref[...]
        x = nrm[:, 0:1]
        y = nrm[:, 1:2]
        z = nrm[:, 2:3]
        sgn = 2.0 * (z >= 0).astype(jnp.float32) - 1.0
        a = -1.0 / (sgn + z)
        b = x * y * a
        u = jnp.concatenate([1.0 + sgn * x * x * a, sgn * b, -sgn * x], axis=1)
        v = jnp.concatenate([b, sgn + y * y * a, -y], axis=1)
        pi = xyzi_ref[...] * inv
        vv = acc_ref[:, 0:3] - pi * acc_ref[:, 3:4]
        ov0 = jnp.sum(u * vv, axis=1, keepdims=True) + 1e-5
        ov1 = jnp.sum(v * vv, axis=1, keepdims=True) + 1e-5
        inv_n = 1.0 / jnp.maximum(jnp.sqrt(ov0 * ov0 + ov1 * ov1), 1e-12)
        ex = ov0 * inv_n
        ey = ov1 * inv_n
        u2 = ex * u + ey * v
        v2 = -ey * u + ex * v
        bas9 = jnp.concatenate([nrm, u2, v2], axis=1)
        c_ref[...] = jnp.dot(bas9, m924_ref[...],
                             preferred_element_type=jnp.float32)


def _pass2_body(nj_steps, xyzt_ref, xyzi_ref, nrmt_ref, nrmi_ref, c_ref,
                f_ref, b1_ref, a2t_ref, b2_ref, wn1t_ref, bn1_ref,
                wn2t_ref, bn2_ref, y_ref, acc_ref):
    j = pl.program_id(1)

    @pl.when(j == 0)
    def _():
        acc_ref[...] = jnp.zeros_like(acc_ref)

    inv = 1.0 / (SQRT2 * RADIUS)
    pxj = xyzt_ref[0:1, :] * inv
    pyj = xyzt_ref[1:2, :] * inv
    pzj = xyzt_ref[2:3, :] * inv
    pxi = xyzi_ref[:, 0:1] * inv
    pyi = xyzi_ref[:, 1:2] * inv
    pzi = xyzi_ref[:, 2:3] * inv
    dx = pxj - pxi
    dy = pyj - pyi
    dz = pzj - pzi
    d2 = dx * dx + dy * dy + dz * dz
    cos = jnp.dot(nrmi_ref[...], nrmt_ref[...],
                  preferred_element_type=jnp.float32)
    t = 2.0 - cos
    win = jnp.exp(-d2 * t * t)
    bf16 = jnp.bfloat16
    fb = f_ref[...]
    fb16 = fb.astype(bf16)
    win16 = win.astype(bf16)
    dx16 = dx.astype(bf16)
    dy16 = dy.astype(bf16)
    dz16 = dz.astype(bf16)
    cb = c_ref[...].astype(bf16)
    b1b = b1_ref[...].astype(bf16)
    acc = acc_ref[...]
    acc = acc + (jnp.dot(win, fb,
                         preferred_element_type=jnp.float32) * b2_ref[...])
    for c in range(CUTS):
        cx = cb[:, 3 * c:3 * c + 1]
        cy = cb[:, 3 * c + 1:3 * c + 2]
        cz = cb[:, 3 * c + 2:3 * c + 3]
        p = dx16 * cx + dy16 * cy + dz16 * cz + b1b[0:1, c:c + 1]
        wc = win16 * jnp.maximum(p, jnp.zeros((), bf16))
        acc = acc + (jnp.dot(wc, fb16, preferred_element_type=jnp.float32)
                     * a2t_ref[c:c + 1, :])
    acc_ref[...] = acc

    @pl.when(j == nj_steps - 1)
    def _():
        out = _leaky(jnp.dot(acc_ref[...], wn1t_ref[...]) + bn1_ref[...])
        out = _leaky(jnp.dot(out, wn2t_ref[...]) + bn2_ref[...])
        y_ref[...] = out


def _post_body(y_ref, feats_ref, gout_ref, bout_ref, wl1t_ref, bl1_ref,
               wl2t_ref, bl2_ref, wtt_ref, bt_ref, out_ref):
    yn = _group_norm_cols(y_ref[...], gout_ref[...], bout_ref[...], float(N))
    h = jnp.maximum(jnp.dot(yn, wl1t_ref[...]) + bl1_ref[...], 0.0)
    h = jnp.dot(h, wl2t_ref[...]) + bl2_ref[...]
    out_ref[...] = jnp.dot(feats_ref[...], wtt_ref[...]) + bt_ref[...] + h


def _pad_rows(x, value):
    return jnp.pad(x, ((0, NPAD - x.shape[0]), (0, 0)), constant_values=value)


@jax.jit
def kernel(surface_xyz, surface_normals, features, Wo1, bo1, Wo2, bo2,
           Wi1, bi1, Wi2, bi2, g_in, b_in, A1, B1, A2, B2,
           Wn1, bn1, Wn2, bn2, g_out, b_out, Wl1, bl1, Wl2, bl2, Wt, bt):
    f32 = jnp.float32
    row = lambda b: b.reshape(1, -1)

    scores, f = pl.pallas_call(
        _pre_body,
        out_shape=(jax.ShapeDtypeStruct((N, 1), f32),
                   jax.ShapeDtypeStruct((N, HID), f32)),
    )(features, Wo1.T, row(bo1), Wo2.T, row(bo2),
      Wi1.T, row(bi1), Wi2.T, row(bi2), row(g_in), row(b_in))

    xyz_p = _pad_rows(surface_xyz, 1e6)
    nrm_p = _pad_rows(surface_normals, 0.0)
    f_p = _pad_rows(f, 0.0)
    w_p = _pad_rows(scores, 0.0)
    xyz_t = xyz_p.T
    nrm_t = nrm_p.T
    w_t = w_p.T

    ni = NPAD // CI
    nj = NPAD // CJ

    m924 = jnp.zeros((9, 3 * CUTS), f32)
    for k in range(3):
        for d in range(3):
            m924 = m924.at[3 * k + d, d::3].set(A1[:, k])

    c24 = pl.pallas_call(
        functools.partial(_pass1_body, nj),
        grid=(ni, nj),
        in_specs=[
            pl.BlockSpec((3, CJ), lambda i, j: (0, j)),
            pl.BlockSpec((CI, 3), lambda i, j: (i, 0)),
            pl.BlockSpec((CJ, 3), lambda i, j: (j, 0)),
            pl.BlockSpec((3, CJ), lambda i, j: (0, j)),
            pl.BlockSpec((CI, 3), lambda i, j: (i, 0)),
            pl.BlockSpec((1, CJ), lambda i, j: (0, j)),
            pl.BlockSpec((9, 3 * CUTS), lambda i, j: (0, 0)),
        ],
        out_specs=pl.BlockSpec((CI, 3 * CUTS), lambda i, j: (i, 0)),
        out_shape=jax.ShapeDtypeStruct((NPAD, 3 * CUTS), f32),
        scratch_shapes=[pltpu.VMEM((CI, 4), f32)],
        compiler_params=pltpu.CompilerParams(
            dimension_semantics=("arbitrary", "arbitrary")),
    )(xyz_t, xyz_p, xyz_p, nrm_t, nrm_p, w_t, m924)

    y = pl.pallas_call(
        functools.partial(_pass2_body, nj),
        grid=(ni, nj),
        in_specs=[
            pl.BlockSpec((3, CJ), lambda i, j: (0, j)),
            pl.BlockSpec((CI, 3), lambda i, j: (i, 0)),
            pl.BlockSpec((3, CJ), lambda i, j: (0, j)),
            pl.BlockSpec((CI, 3), lambda i, j: (i, 0)),
            pl.BlockSpec((CI, 3 * CUTS), lambda i, j: (i, 0)),
            pl.BlockSpec((CJ, HID), lambda i, j: (j, 0)),
            pl.BlockSpec((1, CUTS), lambda i, j: (0, 0)),
            pl.BlockSpec((CUTS, HID), lambda i, j: (0, 0)),
            pl.BlockSpec((1, HID), lambda i, j: (0, 0)),
            pl.BlockSpec((HID, HID), lambda i, j: (0, 0)),
            pl.BlockSpec((1, HID), lambda i, j: (0, 0)),
            pl.BlockSpec((HID, HID), lambda i, j: (0, 0)),
            pl.BlockSpec((1, HID), lambda i, j: (0, 0)),
        ],
        out_specs=pl.BlockSpec((CI, HID), lambda i, j: (i, 0)),
        out_shape=jax.ShapeDtypeStruct((NPAD, HID), f32),
        scratch_shapes=[pltpu.VMEM((CI, HID), f32)],
        compiler_params=pltpu.CompilerParams(
            dimension_semantics=("arbitrary", "arbitrary")),
    )(xyz_t, xyz_p, nrm_t, nrm_p, c24, f_p, row(B1), A2[:HID].T,
      row(B2[:HID]), Wn1.T, row(bn1), Wn2.T, row(bn2))

    out = pl.pallas_call(
        _post_body,
        out_shape=jax.ShapeDtypeStruct((N, HID), f32),
    )(y[:N], features, row(g_out), row(b_out),
      Wl1.T, row(bl1), Wl2.T, row(bl2), Wt.T, row(bt))
    return out
